# baseline (device time: 41434 ns/iter reference)
import os

import jax
import jax.numpy as jnp
from jax import lax
from jax.experimental import pallas as pl
from jax.experimental.pallas import tpu as pltpu

_ABLATE = os.environ.get("ABLATE", "")
_DO_DATA = _ABLATE not in ("nocomm", "dmaonly")
_DO_AMAX = _ABLATE not in ("nocomm", "noamax", "dmaonly")
_DO_MATH = _ABLATE != "dmaonly"

N_DEV = 32
N_CHUNKS = 8
DEV_PER_CHUNK = N_DEV // N_CHUNKS


def kernel(x, w_mat):
    m_per, k = x.shape
    _, n = w_mat.shape
    n_per = n // N_DEV
    n_ck = n // N_CHUNKS

    def body(x_ref, w_hbm, out_ref,
             w_vmem, y_src, recv2d, amax_src, amax_recv,
             w_sems, send_sems, recv_sems, am_send_sems, am_recv_sems):
        my = lax.axis_index("i")

        grp = my // DEV_PER_CHUNK

        with jax.named_scope("w_dma_issue"):
            for c in range(N_CHUNKS):
                c_eff = lax.rem(c + grp, N_CHUNKS)
                pltpu.make_async_copy(
                    w_hbm.at[:, pl.ds(c_eff * n_ck, n_ck)],
                    w_vmem.at[c],
                    w_sems.at[c],
                ).start()

        with jax.named_scope("x_cast"):
            xb = x_ref[:, :].astype(jnp.bfloat16)

        amax = jnp.float32(0.0)
        for c in range(N_CHUNKS):
            c_eff = lax.rem(c + grp, N_CHUNKS)
            with jax.named_scope(f"w_wait#{c}"):
                pltpu.make_async_copy(
                    w_hbm.at[:, pl.ds(c_eff * n_ck, n_ck)],
                    w_vmem.at[c],
                    w_sems.at[c],
                ).wait()
            if _DO_MATH:
                with jax.named_scope(f"mm#{c}"):
                    wb = w_vmem[c].astype(jnp.bfloat16)
                    yc = jnp.dot(xb, wb,
                                 preferred_element_type=jnp.float32)
                    amax = jnp.maximum(amax, jnp.max(jnp.abs(yc)))
                    ycb = yc.astype(jnp.bfloat16)
            else:
                amax = jnp.maximum(amax, jnp.max(jnp.abs(w_vmem[c, 0, :])))
                ycb = None
            with jax.named_scope(f"store_send#{c}"):
                for t in range(DEV_PER_CHUNK):
                    j = c_eff * DEV_PER_CHUNK + t
                    if _DO_MATH:
                        y_src[pl.ds(j, 1), :, :] = (
                            ycb[:, t * n_per:(t + 1) * n_per][None])

                    @pl.when(j == my)
                    def _(j=j):
                        recv2d[pl.ds(my * m_per, m_per), :] = (
                            y_src[pl.ds(j, 1), :, :][0])

                    if _DO_DATA:
                        @pl.when(j != my)
                        def _(j=j):
                            pltpu.make_async_remote_copy(
                                src_ref=y_src.at[j],
                                dst_ref=recv2d.at[pl.ds(my * m_per, m_per), :],
                                send_sem=send_sems.at[j],
                                recv_sem=recv_sems.at[my],
                                device_id=(j,),
                                device_id_type=pl.DeviceIdType.MESH,
                            ).start()

        with jax.named_scope("amax_send"):
            amax_src[0, :] = jnp.full((128,), amax, dtype=jnp.float32)
            amax_recv[my, :] = jnp.full((128,), amax, dtype=jnp.float32)
            if _DO_AMAX:
                for i in range(N_DEV - 1):
                    j = lax.rem(my + 1 + i, N_DEV)
                    pltpu.make_async_remote_copy(
                        src_ref=amax_src,
                        dst_ref=amax_recv.at[pl.ds(my, 1), :],
                        send_sem=am_send_sems.at[j],
                        recv_sem=am_recv_sems.at[my],
                        device_id=(j,),
                        device_id_type=pl.DeviceIdType.MESH,
                    ).start()

        with jax.named_scope("amax_wait"):
            if _DO_AMAX:
                for s in range(N_DEV):
                    @pl.when(s != my)
                    def _(s=s):
                        pltpu.make_async_remote_copy(
                            src_ref=amax_src,
                            dst_ref=amax_recv.at[pl.ds(s, 1), :],
                            send_sem=am_send_sems.at[s],
                            recv_sem=am_recv_sems.at[s],
                            device_id=(my,),
                            device_id_type=pl.DeviceIdType.MESH,
                        ).wait_recv()
                g_amax = jnp.max(amax_recv[:, :])
            else:
                g_amax = amax
        inv_scale = 127.0 / g_amax
        scale = g_amax / 127.0

        with jax.named_scope("data_wait"):
            for s in range(N_DEV) if _DO_DATA else ():
                @pl.when(s != my)
                def _(s=s):
                    pltpu.make_async_remote_copy(
                        src_ref=y_src.at[s],
                        dst_ref=recv2d.at[pl.ds(s * m_per, m_per), :],
                        send_sem=send_sems.at[s],
                        recv_sem=recv_sems.at[s],
                        device_id=(my,),
                        device_id_type=pl.DeviceIdType.MESH,
                    ).wait_recv()

        with jax.named_scope("quant"):
            yf = recv2d[:, :].astype(jnp.float32)
            q = jnp.clip(jnp.round(yf * inv_scale), -127.0, 127.0)
            out_ref[:, :] = q * scale

        with jax.named_scope("drain"):
            for j in range(N_DEV):
                if _DO_DATA:
                    @pl.when(j != my)
                    def _(j=j):
                        pltpu.make_async_remote_copy(
                            src_ref=y_src.at[j],
                            dst_ref=recv2d.at[pl.ds(my * m_per, m_per), :],
                            send_sem=send_sems.at[j],
                            recv_sem=recv_sems.at[my],
                            device_id=(j,),
                            device_id_type=pl.DeviceIdType.MESH,
                        ).wait_send()
                if _DO_AMAX:
                    @pl.when(j != my)
                    def _(j=j):
                        pltpu.make_async_remote_copy(
                            src_ref=amax_src,
                            dst_ref=amax_recv.at[pl.ds(my, 1), :],
                            send_sem=am_send_sems.at[j],
                            recv_sem=am_recv_sems.at[my],
                            device_id=(j,),
                            device_id_type=pl.DeviceIdType.MESH,
                        ).wait_send()

    return pl.pallas_call(
        body,
        out_shape=jax.ShapeDtypeStruct((N_DEV * m_per, n_per), jnp.float32),
        in_specs=[
            pl.BlockSpec(memory_space=pltpu.VMEM),
            pl.BlockSpec(memory_space=pltpu.MemorySpace.HBM),
        ],
        out_specs=pl.BlockSpec(memory_space=pltpu.VMEM),
        scratch_shapes=[
            pltpu.VMEM((N_CHUNKS, k, n // N_CHUNKS), jnp.float32),
            pltpu.VMEM((N_DEV, m_per, n_per), jnp.bfloat16),
            pltpu.VMEM((N_DEV * m_per, n_per), jnp.bfloat16),
            pltpu.VMEM((1, 128), jnp.float32),
            pltpu.VMEM((N_DEV, 128), jnp.float32),
            pltpu.SemaphoreType.DMA((N_CHUNKS,)),
            pltpu.SemaphoreType.DMA((N_DEV,)),
            pltpu.SemaphoreType.DMA((N_DEV,)),
            pltpu.SemaphoreType.DMA((N_DEV,)),
            pltpu.SemaphoreType.DMA((N_DEV,)),
        ],
        compiler_params=pltpu.CompilerParams(
            vmem_limit_bytes=100 * 1024 * 1024,
        ),
    )(x, w_mat)


# device time: 34047 ns/iter; 1.2170x vs baseline; 1.2170x over previous
import os

import jax
import jax.numpy as jnp
from jax import lax
from jax.experimental import pallas as pl
from jax.experimental.pallas import tpu as pltpu

_ABLATE = os.environ.get("ABLATE", "")
_DO_DATA = _ABLATE not in ("nocomm", "dmaonly")
_DO_AMAX = _ABLATE not in ("nocomm", "noamax", "dmaonly")
_DO_MATH = _ABLATE != "dmaonly"

N_DEV = 32
N_CHUNKS = 8
DEV_PER_CHUNK = N_DEV // N_CHUNKS


def kernel(x, w_mat):
    m_per, k = x.shape
    _, n = w_mat.shape
    n_per = n // N_DEV
    n_ck = n // N_CHUNKS

    def body(x_ref, w_hbm, out_ref,
             w_vmem, y_src, recv2d, amax_src, amax_recv,
             w_sems, send_sems, recv_sems, am_send_sems, am_recv_sems):
        my = lax.axis_index("i")

        with jax.named_scope("barrier"):
            barrier_sem = pltpu.get_barrier_semaphore()
            left = lax.rem(my + N_DEV - 1, N_DEV)
            right = lax.rem(my + 1, N_DEV)
            pl.semaphore_signal(barrier_sem, inc=1, device_id=(left,),
                                device_id_type=pl.DeviceIdType.MESH)
            pl.semaphore_signal(barrier_sem, inc=1, device_id=(right,),
                                device_id_type=pl.DeviceIdType.MESH)
            pl.semaphore_wait(barrier_sem, 2)

        grp = my // DEV_PER_CHUNK

        with jax.named_scope("w_dma_issue"):
            for c in range(N_CHUNKS):
                c_eff = lax.rem(c + grp, N_CHUNKS)
                pltpu.make_async_copy(
                    w_hbm.at[:, pl.ds(c_eff * n_ck, n_ck)],
                    w_vmem.at[c],
                    w_sems.at[c],
                ).start()

        with jax.named_scope("x_cast"):
            xb = x_ref[:, :].astype(jnp.bfloat16)

        amax = jnp.float32(0.0)
        for c in range(N_CHUNKS):
            c_eff = lax.rem(c + grp, N_CHUNKS)
            with jax.named_scope(f"w_wait#{c}"):
                pltpu.make_async_copy(
                    w_hbm.at[:, pl.ds(c_eff * n_ck, n_ck)],
                    w_vmem.at[c],
                    w_sems.at[c],
                ).wait()
            if _DO_MATH:
                with jax.named_scope(f"mm#{c}"):
                    wb = w_vmem[c].astype(jnp.bfloat16)
                    yc = jnp.dot(xb, wb,
                                 preferred_element_type=jnp.float32)
                    amax = jnp.maximum(amax, jnp.max(jnp.abs(yc)))
                    ycb = yc.astype(jnp.bfloat16)
            else:
                amax = jnp.maximum(amax, jnp.max(jnp.abs(w_vmem[c, 0, :])))
                ycb = None
            with jax.named_scope(f"store_send#{c}"):
                for t in range(DEV_PER_CHUNK):
                    j = c_eff * DEV_PER_CHUNK + t
                    if _DO_MATH:
                        y_src[pl.ds(j, 1), :, :] = (
                            ycb[:, t * n_per:(t + 1) * n_per][None])

                    @pl.when(j == my)
                    def _(j=j):
                        recv2d[pl.ds(my * m_per, m_per), :] = (
                            y_src[pl.ds(j, 1), :, :][0])

                    if _DO_DATA:
                        @pl.when(j != my)
                        def _(j=j):
                            pltpu.make_async_remote_copy(
                                src_ref=y_src.at[j],
                                dst_ref=recv2d.at[pl.ds(my * m_per, m_per), :],
                                send_sem=send_sems.at[j],
                                recv_sem=recv_sems.at[my],
                                device_id=(j,),
                                device_id_type=pl.DeviceIdType.MESH,
                            ).start()

        with jax.named_scope("amax_send"):
            amax_src[0, :] = jnp.full((128,), amax, dtype=jnp.float32)
            amax_recv[my, :] = jnp.full((128,), amax, dtype=jnp.float32)
            if _DO_AMAX:
                for i in range(N_DEV - 1):
                    j = lax.rem(my + 1 + i, N_DEV)
                    pltpu.make_async_remote_copy(
                        src_ref=amax_src,
                        dst_ref=amax_recv.at[pl.ds(my, 1), :],
                        send_sem=am_send_sems.at[j],
                        recv_sem=am_recv_sems.at[my],
                        device_id=(j,),
                        device_id_type=pl.DeviceIdType.MESH,
                    ).start()

        with jax.named_scope("amax_wait"):
            if _DO_AMAX:
                for s in range(N_DEV):
                    @pl.when(s != my)
                    def _(s=s):
                        pltpu.make_async_remote_copy(
                            src_ref=amax_src,
                            dst_ref=amax_recv.at[pl.ds(s, 1), :],
                            send_sem=am_send_sems.at[s],
                            recv_sem=am_recv_sems.at[s],
                            device_id=(my,),
                            device_id_type=pl.DeviceIdType.MESH,
                        ).wait_recv()
                g_amax = jnp.max(amax_recv[:, :])
            else:
                g_amax = amax
        inv_scale = 127.0 / g_amax
        scale = g_amax / 127.0

        with jax.named_scope("data_wait"):
            for s in range(N_DEV) if _DO_DATA else ():
                @pl.when(s != my)
                def _(s=s):
                    pltpu.make_async_remote_copy(
                        src_ref=y_src.at[s],
                        dst_ref=recv2d.at[pl.ds(s * m_per, m_per), :],
                        send_sem=send_sems.at[s],
                        recv_sem=recv_sems.at[s],
                        device_id=(my,),
                        device_id_type=pl.DeviceIdType.MESH,
                    ).wait_recv()

        with jax.named_scope("quant"):
            yf = recv2d[:, :].astype(jnp.float32)
            q = jnp.clip(jnp.round(yf * inv_scale), -127.0, 127.0)
            out_ref[:, :] = q * scale

        with jax.named_scope("drain"):
            for j in range(N_DEV):
                if _DO_DATA:
                    @pl.when(j != my)
                    def _(j=j):
                        pltpu.make_async_remote_copy(
                            src_ref=y_src.at[j],
                            dst_ref=recv2d.at[pl.ds(my * m_per, m_per), :],
                            send_sem=send_sems.at[j],
                            recv_sem=recv_sems.at[my],
                            device_id=(j,),
                            device_id_type=pl.DeviceIdType.MESH,
                        ).wait_send()
                if _DO_AMAX:
                    @pl.when(j != my)
                    def _(j=j):
                        pltpu.make_async_remote_copy(
                            src_ref=amax_src,
                            dst_ref=amax_recv.at[pl.ds(my, 1), :],
                            send_sem=am_send_sems.at[j],
                            recv_sem=am_recv_sems.at[my],
                            device_id=(j,),
                            device_id_type=pl.DeviceIdType.MESH,
                        ).wait_send()

    return pl.pallas_call(
        body,
        out_shape=jax.ShapeDtypeStruct((N_DEV * m_per, n_per), jnp.float32),
        in_specs=[
            pl.BlockSpec(memory_space=pltpu.VMEM),
            pl.BlockSpec(memory_space=pltpu.MemorySpace.HBM),
        ],
        out_specs=pl.BlockSpec(memory_space=pltpu.VMEM),
        scratch_shapes=[
            pltpu.VMEM((N_CHUNKS, k, n // N_CHUNKS), jnp.float32),
            pltpu.VMEM((N_DEV, m_per, n_per), jnp.bfloat16),
            pltpu.VMEM((N_DEV * m_per, n_per), jnp.bfloat16),
            pltpu.VMEM((1, 128), jnp.float32),
            pltpu.VMEM((N_DEV, 128), jnp.float32),
            pltpu.SemaphoreType.DMA((N_CHUNKS,)),
            pltpu.SemaphoreType.DMA((N_DEV,)),
            pltpu.SemaphoreType.DMA((N_DEV,)),
            pltpu.SemaphoreType.DMA((N_DEV,)),
            pltpu.SemaphoreType.DMA((N_DEV,)),
        ],
        compiler_params=pltpu.CompilerParams(
            vmem_limit_bytes=100 * 1024 * 1024,
            collective_id=0,
        ),
    )(x, w_mat)
